# baseline (speedup 1.0000x reference)
import functools

import jax
import jax.numpy as jnp
from jax import lax
from jax.experimental import pallas as pl
from jax.experimental.pallas import tpu as pltpu
from jax.experimental.pallas import tpu_sc as plsc

N_NODES = 10000
N_EDGES = 320000
F_IN = 128
H_ENC = 32

NC, NS, LANES = 2, 16, 16
NW = NC * NS
CH = 100
NCH = 100
KOUT = 10
ACC_ROWS = 10240
RPT = ACC_ROWS // NS
ZR = 64
ONES_PAD = 112

BM = 2000


def _encode(x, W_enc):
    def body(x_ref, w_ref, o_ref):
        o_ref[...] = jnp.dot(x_ref[...], w_ref[...], preferred_element_type=jnp.float32)

    return pl.pallas_call(
        body,
        grid=(N_NODES // BM,),
        in_specs=[
            pl.BlockSpec((BM, F_IN), lambda i: (i, 0)),
            pl.BlockSpec((F_IN, H_ENC), lambda i: (0, 0)),
        ],
        out_specs=pl.BlockSpec((BM, H_ENC), lambda i: (i, 0)),
        out_shape=jax.ShapeDtypeStruct((N_NODES, H_ENC), jnp.float32),
    )(x, W_enc)


def _sc_segment_sum(y, srcs, dsts):
    mesh = plsc.VectorSubcoreMesh(core_axis_name="c", subcore_axis_name="s")

    @functools.partial(
        pl.kernel,
        out_type=(
            jax.ShapeDtypeStruct((NC, ACC_ROWS, H_ENC), jnp.float32),
            jax.ShapeDtypeStruct((NC, ACC_ROWS), jnp.float32),
        ),
        mesh=mesh,
        scratch_types=[
            pltpu.VMEM((NCH, CH), jnp.int32),
            pltpu.VMEM((NCH, CH), jnp.int32),
            pltpu.VMEM((KOUT, CH, H_ENC), jnp.float32),
            pltpu.VMEM((ZR, H_ENC), jnp.float32),
            pltpu.VMEM((RPT,), jnp.float32),
            pltpu.VMEM((ONES_PAD,), jnp.float32),
            pltpu.VMEM_SHARED((N_NODES, H_ENC), jnp.float32),
            pltpu.VMEM_SHARED((ACC_ROWS, H_ENC), jnp.float32),
            pltpu.VMEM_SHARED((ACC_ROWS,), jnp.float32),
            pltpu.SemaphoreType.DMA,
            pltpu.SemaphoreType.DMA,
            pltpu.SemaphoreType.DMA,
        ],
        compiler_params=pltpu.CompilerParams(use_tc_tiling_on_sc=False),
    )
    def run(y_hbm, src_hbm, dst_hbm, out_hbm, deg_hbm, src_v, dst_v, rows_v,
            zb_v, dz_v, ones_v, y_sh, acc_sh, deg_sh, sem_a, sem_b, sem_c):
        c = lax.axis_index("c")
        s = lax.axis_index("s")
        wid = s * NC + c

        pend = [
            pltpu.async_copy(src_hbm.at[wid], src_v, sem_a),
            pltpu.async_copy(dst_hbm.at[wid], dst_v, sem_a),
        ]
        ypt = N_NODES // NS
        pend.append(
            pltpu.async_copy(
                y_hbm.at[pl.ds(s * ypt, ypt)], y_sh.at[pl.ds(s * ypt, ypt)], sem_b
            )
        )
        zeros16 = jnp.zeros((LANES,), jnp.float32)
        ones16 = jnp.full((LANES,), 1.0, jnp.float32)
        for r in range(ZR):
            for g in range(H_ENC // LANES):
                zb_v[r, pl.ds(g * LANES, LANES)] = zeros16
        for r in range(RPT // LANES):
            dz_v[pl.ds(r * LANES, LANES)] = zeros16
        for r in range(ONES_PAD // LANES):
            ones_v[pl.ds(r * LANES, LANES)] = ones16
        for b in range(RPT // ZR):
            pend.append(
                pltpu.async_copy(
                    zb_v, acc_sh.at[pl.ds(s * RPT + b * ZR, ZR)], sem_c
                )
            )
        pend.append(pltpu.async_copy(dz_v, deg_sh.at[pl.ds(s * RPT, RPT)], sem_c))
        for d in pend:
            d.wait()
        plsc.subcore_barrier()

        def group(g, carry):
            j0 = g * KOUT
            gs = [
                pltpu.async_copy(y_sh.at[src_v.at[j0 + k]], rows_v.at[k], sem_a)
                for k in range(KOUT)
            ]
            for d in gs:
                d.wait()
            ss = [
                pltpu.async_copy(
                    rows_v.at[k], acc_sh.at[dst_v.at[j0 + k]], sem_b, add=True
                )
                for k in range(KOUT)
            ]
            ds_ = [
                pltpu.async_copy(
                    ones_v.at[pl.ds(0, CH)], deg_sh.at[dst_v.at[j0 + k]],
                    sem_c, add=True
                )
                for k in range(KOUT)
            ]
            for d in ss:
                d.wait()
            for d in ds_:
                d.wait()
            return carry

        lax.fori_loop(0, NCH // KOUT, group, 0)
        plsc.subcore_barrier()

        wb = [
            pltpu.async_copy(
                acc_sh.at[pl.ds(s * RPT, RPT)],
                out_hbm.at[c, pl.ds(s * RPT, RPT)],
                sem_a,
            ),
            pltpu.async_copy(
                deg_sh.at[pl.ds(s * RPT, RPT)],
                deg_hbm.at[c, pl.ds(s * RPT, RPT)],
                sem_b,
            ),
        ]
        for d in wb:
            d.wait()

    return run(y, srcs, dsts)


def _head(parts, degs, b_enc, W1, b1, W2, b2, W3, b3):
    def leaky(t):
        return jnp.where(t >= 0, t, 0.01 * t)

    def body(p_ref, d_ref, be_ref, w1_ref, b1_ref, w2_ref, b2_ref, w3_ref,
             b3_ref, o_ref):
        t = p_ref[0] + p_ref[1]
        deg = jnp.maximum(d_ref[0] + d_ref[1], 1.0)
        agg = t / deg
        h = leaky(agg + be_ref[...])
        z = leaky(jnp.dot(h, w1_ref[...], preferred_element_type=jnp.float32) + b1_ref[...])
        z = leaky(jnp.dot(z, w2_ref[...], preferred_element_type=jnp.float32) + b2_ref[...])
        o_ref[...] = jnp.dot(z, w3_ref[...], preferred_element_type=jnp.float32) + b3_ref[...]

    H, C = W2.shape[0], W3.shape[1]
    return pl.pallas_call(
        body,
        grid=(N_NODES // BM,),
        in_specs=[
            pl.BlockSpec((NC, BM, H_ENC), lambda i: (0, i, 0)),
            pl.BlockSpec((NC, BM, 1), lambda i: (0, i, 0)),
            pl.BlockSpec((1, H_ENC), lambda i: (0, 0)),
            pl.BlockSpec((H_ENC, H), lambda i: (0, 0)),
            pl.BlockSpec((1, H), lambda i: (0, 0)),
            pl.BlockSpec((H, H), lambda i: (0, 0)),
            pl.BlockSpec((1, H), lambda i: (0, 0)),
            pl.BlockSpec((H, C), lambda i: (0, 0)),
            pl.BlockSpec((1, C), lambda i: (0, 0)),
        ],
        out_specs=pl.BlockSpec((BM, C), lambda i: (i, 0)),
        out_shape=jax.ShapeDtypeStruct((N_NODES, C), jnp.float32),
    )(parts, degs, b_enc, W1, b1, W2, b2, W3, b3)


def kernel(x, edge_index, W_enc, b_enc, W1, b1, W2, b2, W3, b3):
    y = _encode(x, W_enc)

    srcs = edge_index[0].reshape(NW, NCH, CH)
    dsts = edge_index[1].reshape(NW, NCH, CH)
    parts, degs = _sc_segment_sum(y, srcs, dsts)

    return _head(
        parts,
        degs.reshape(NC, ACC_ROWS, 1),
        b_enc.reshape(1, -1),
        W1,
        b1.reshape(1, -1),
        W2,
        b2.reshape(1, -1),
        W3,
        b3.reshape(1, -1),
    )


# speedup vs baseline: 1.0112x; 1.0112x over previous
import functools

import jax
import jax.numpy as jnp
from jax import lax
from jax.experimental import pallas as pl
from jax.experimental.pallas import tpu as pltpu
from jax.experimental.pallas import tpu_sc as plsc

N_NODES = 10000
N_EDGES = 320000
F_IN = 128
H_ENC = 32

NC, NS, LANES = 2, 16, 16
NW = NC * NS
CH = 80
NCH = 125
EPW = NCH * CH
KOUT = 5
ACC_ROWS = 10240
RPT = ACC_ROWS // NS
ZR = 64
ONES_PAD = 112

BM = 2000


def _encode(x, W_enc):
    def body(x_ref, w_ref, o_ref):
        o_ref[...] = jnp.dot(x_ref[...], w_ref[...], preferred_element_type=jnp.float32)

    return pl.pallas_call(
        body,
        grid=(N_NODES // BM,),
        in_specs=[
            pl.BlockSpec((BM, F_IN), lambda i: (i, 0)),
            pl.BlockSpec((F_IN, H_ENC), lambda i: (0, 0)),
        ],
        out_specs=pl.BlockSpec((BM, H_ENC), lambda i: (i, 0)),
        out_shape=jax.ShapeDtypeStruct((N_NODES, H_ENC), jnp.float32),
    )(x, W_enc)


def _sc_segment_sum(y, edge_index):
    mesh = plsc.VectorSubcoreMesh(core_axis_name="c", subcore_axis_name="s")

    @functools.partial(
        pl.kernel,
        out_type=(
            jax.ShapeDtypeStruct((NC, ACC_ROWS, H_ENC), jnp.float32),
            jax.ShapeDtypeStruct((NC, ACC_ROWS), jnp.float32),
        ),
        mesh=mesh,
        scratch_types=[
            pltpu.VMEM((EPW,), jnp.int32),
            pltpu.VMEM((EPW,), jnp.int32),
            pltpu.VMEM((KOUT, CH, H_ENC), jnp.float32),
            pltpu.VMEM((ZR, H_ENC), jnp.float32),
            pltpu.VMEM((RPT,), jnp.float32),
            pltpu.VMEM((ONES_PAD,), jnp.float32),
            pltpu.VMEM_SHARED((N_NODES, H_ENC), jnp.float32),
            pltpu.VMEM_SHARED((ACC_ROWS, H_ENC), jnp.float32),
            pltpu.VMEM_SHARED((ACC_ROWS,), jnp.float32),
            pltpu.SemaphoreType.DMA,
            pltpu.SemaphoreType.DMA,
            pltpu.SemaphoreType.DMA,
        ],
        compiler_params=pltpu.CompilerParams(use_tc_tiling_on_sc=False),
    )
    def run(y_hbm, edge_hbm, out_hbm, deg_hbm, src_v, dst_v, rows_v,
            zb_v, dz_v, ones_v, y_sh, acc_sh, deg_sh, sem_a, sem_b, sem_c):
        c = lax.axis_index("c")
        s = lax.axis_index("s")
        wid = s * NC + c

        pend = [
            pltpu.async_copy(edge_hbm.at[0, pl.ds(wid * EPW, EPW)], src_v, sem_a),
            pltpu.async_copy(edge_hbm.at[1, pl.ds(wid * EPW, EPW)], dst_v, sem_a),
        ]
        ypt = N_NODES // NS
        pend.append(
            pltpu.async_copy(
                y_hbm.at[pl.ds(s * ypt, ypt)], y_sh.at[pl.ds(s * ypt, ypt)], sem_b
            )
        )
        zeros16 = jnp.zeros((LANES,), jnp.float32)
        ones16 = jnp.full((LANES,), 1.0, jnp.float32)
        for r in range(ZR):
            for g in range(H_ENC // LANES):
                zb_v[r, pl.ds(g * LANES, LANES)] = zeros16
        for r in range(RPT // LANES):
            dz_v[pl.ds(r * LANES, LANES)] = zeros16
        for r in range(ONES_PAD // LANES):
            ones_v[pl.ds(r * LANES, LANES)] = ones16
        for b in range(RPT // ZR):
            pend.append(
                pltpu.async_copy(
                    zb_v, acc_sh.at[pl.ds(s * RPT + b * ZR, ZR)], sem_c
                )
            )
        pend.append(pltpu.async_copy(dz_v, deg_sh.at[pl.ds(s * RPT, RPT)], sem_c))
        for d in pend:
            d.wait()
        plsc.subcore_barrier()

        def group(g, carry):
            j0 = g * KOUT
            gs = [
                pltpu.async_copy(
                    y_sh.at[src_v.at[pl.ds((j0 + k) * CH, CH)]], rows_v.at[k], sem_a
                )
                for k in range(KOUT)
            ]
            for d in gs:
                d.wait()
            ss = [
                pltpu.async_copy(
                    rows_v.at[k], acc_sh.at[dst_v.at[pl.ds((j0 + k) * CH, CH)]],
                    sem_b, add=True
                )
                for k in range(KOUT)
            ]
            ds_ = [
                pltpu.async_copy(
                    ones_v.at[pl.ds(0, CH)],
                    deg_sh.at[dst_v.at[pl.ds((j0 + k) * CH, CH)]],
                    sem_c, add=True
                )
                for k in range(KOUT)
            ]
            for d in ss:
                d.wait()
            for d in ds_:
                d.wait()
            return carry

        lax.fori_loop(0, NCH // KOUT, group, 0)
        plsc.subcore_barrier()

        wb = [
            pltpu.async_copy(
                acc_sh.at[pl.ds(s * RPT, RPT)],
                out_hbm.at[c, pl.ds(s * RPT, RPT)],
                sem_a,
            ),
            pltpu.async_copy(
                deg_sh.at[pl.ds(s * RPT, RPT)],
                deg_hbm.at[c, pl.ds(s * RPT, RPT)],
                sem_b,
            ),
        ]
        for d in wb:
            d.wait()

    return run(y, edge_index)


def _head(parts, degs, b_enc, W1, b1, W2, b2, W3, b3):
    def leaky(t):
        return jnp.where(t >= 0, t, 0.01 * t)

    def body(p_ref, d_ref, be_ref, w1_ref, b1_ref, w2_ref, b2_ref, w3_ref,
             b3_ref, o_ref):
        t = p_ref[0] + p_ref[1]
        deg = jnp.maximum(d_ref[0] + d_ref[1], 1.0)
        agg = t / deg
        h = leaky(agg + be_ref[...])
        z = leaky(jnp.dot(h, w1_ref[...], preferred_element_type=jnp.float32) + b1_ref[...])
        z = leaky(jnp.dot(z, w2_ref[...], preferred_element_type=jnp.float32) + b2_ref[...])
        o_ref[...] = jnp.dot(z, w3_ref[...], preferred_element_type=jnp.float32) + b3_ref[...]

    H, C = W2.shape[0], W3.shape[1]
    return pl.pallas_call(
        body,
        grid=(N_NODES // BM,),
        in_specs=[
            pl.BlockSpec((NC, BM, H_ENC), lambda i: (0, i, 0)),
            pl.BlockSpec((NC, BM, 1), lambda i: (0, i, 0)),
            pl.BlockSpec((H_ENC,), lambda i: (0,)),
            pl.BlockSpec((H_ENC, H), lambda i: (0, 0)),
            pl.BlockSpec((H,), lambda i: (0,)),
            pl.BlockSpec((H, H), lambda i: (0, 0)),
            pl.BlockSpec((H,), lambda i: (0,)),
            pl.BlockSpec((H, C), lambda i: (0, 0)),
            pl.BlockSpec((C,), lambda i: (0,)),
        ],
        out_specs=pl.BlockSpec((BM, C), lambda i: (i, 0)),
        out_shape=jax.ShapeDtypeStruct((N_NODES, C), jnp.float32),
    )(parts, degs, b_enc, W1, b1, W2, b2, W3, b3)


def kernel(x, edge_index, W_enc, b_enc, W1, b1, W2, b2, W3, b3):
    y = _encode(x, W_enc)
    parts, degs = _sc_segment_sum(y, edge_index)
    return _head(parts, degs.reshape(NC, ACC_ROWS, 1), b_enc, W1, b1, W2, b2,
                 W3, b3)


# speedup vs baseline: 1.1245x; 1.1120x over previous
import functools

import jax
import jax.numpy as jnp
from jax import lax
from jax.experimental import pallas as pl
from jax.experimental.pallas import tpu as pltpu
from jax.experimental.pallas import tpu_sc as plsc

N_NODES = 10000
N_EDGES = 320000
F_IN = 128
H_ENC = 32

NC, NS, LANES = 2, 16, 16
NW = NC * NS
CH = 80
NCH = 125
EPW = NCH * CH
KOUT = 12
NTAIL = NCH - (NCH // KOUT) * KOUT
ACC_ROWS = 10240
RPT = ACC_ROWS // NS
ZR = 64
ONES_PAD = 112

BM = 2000


def _encode(x, W_enc):
    def body(x_ref, w_ref, o_ref):
        o_ref[...] = jnp.dot(x_ref[...], w_ref[...], preferred_element_type=jnp.float32)

    return pl.pallas_call(
        body,
        grid=(N_NODES // BM,),
        in_specs=[
            pl.BlockSpec((BM, F_IN), lambda i: (i, 0)),
            pl.BlockSpec((F_IN, H_ENC), lambda i: (0, 0)),
        ],
        out_specs=pl.BlockSpec((BM, H_ENC), lambda i: (i, 0)),
        out_shape=jax.ShapeDtypeStruct((N_NODES, H_ENC), jnp.float32),
    )(x, W_enc)


def _sc_segment_sum(y, edge_index):
    mesh = plsc.VectorSubcoreMesh(core_axis_name="c", subcore_axis_name="s")

    @functools.partial(
        pl.kernel,
        out_type=(
            jax.ShapeDtypeStruct((NC, ACC_ROWS, H_ENC), jnp.float32),
            jax.ShapeDtypeStruct((NC, ACC_ROWS), jnp.float32),
        ),
        mesh=mesh,
        scratch_types=[
            pltpu.VMEM((EPW,), jnp.int32),
            pltpu.VMEM((EPW,), jnp.int32),
            pltpu.VMEM((KOUT, CH, H_ENC), jnp.float32),
            pltpu.VMEM((ZR, H_ENC), jnp.float32),
            pltpu.VMEM((RPT,), jnp.float32),
            pltpu.VMEM((ONES_PAD,), jnp.float32),
            pltpu.VMEM_SHARED((N_NODES, H_ENC), jnp.float32),
            pltpu.VMEM_SHARED((ACC_ROWS, H_ENC), jnp.float32),
            pltpu.VMEM_SHARED((ACC_ROWS,), jnp.float32),
            pltpu.SemaphoreType.DMA,
            pltpu.SemaphoreType.DMA,
            pltpu.SemaphoreType.DMA,
        ],
        compiler_params=pltpu.CompilerParams(use_tc_tiling_on_sc=False),
    )
    def run(y_hbm, edge_hbm, out_hbm, deg_hbm, src_v, dst_v, rows_v,
            zb_v, dz_v, ones_v, y_sh, acc_sh, deg_sh, sem_a, sem_b, sem_c):
        c = lax.axis_index("c")
        s = lax.axis_index("s")
        wid = s * NC + c

        pend = [
            pltpu.async_copy(edge_hbm.at[0, pl.ds(wid * EPW, EPW)], src_v, sem_a),
            pltpu.async_copy(edge_hbm.at[1, pl.ds(wid * EPW, EPW)], dst_v, sem_a),
        ]
        ypt = N_NODES // NS
        pend.append(
            pltpu.async_copy(
                y_hbm.at[pl.ds(s * ypt, ypt)], y_sh.at[pl.ds(s * ypt, ypt)], sem_b
            )
        )
        zeros16 = jnp.zeros((LANES,), jnp.float32)
        ones16 = jnp.full((LANES,), 1.0, jnp.float32)
        for r in range(ZR):
            for g in range(H_ENC // LANES):
                zb_v[r, pl.ds(g * LANES, LANES)] = zeros16
        for r in range(RPT // LANES):
            dz_v[pl.ds(r * LANES, LANES)] = zeros16
        for r in range(ONES_PAD // LANES):
            ones_v[pl.ds(r * LANES, LANES)] = ones16
        for b in range(RPT // ZR):
            pend.append(
                pltpu.async_copy(
                    zb_v, acc_sh.at[pl.ds(s * RPT + b * ZR, ZR)], sem_c
                )
            )
        pend.append(pltpu.async_copy(dz_v, deg_sh.at[pl.ds(s * RPT, RPT)], sem_c))
        for d in pend:
            d.wait()
        plsc.subcore_barrier()

        def run_group(j0, cnt):
            gs = [
                pltpu.async_copy(
                    y_sh.at[src_v.at[pl.ds((j0 + k) * CH, CH)]], rows_v.at[k], sem_a
                )
                for k in range(cnt)
            ]
            for d in gs:
                d.wait()
            ss = [
                pltpu.async_copy(
                    rows_v.at[k], acc_sh.at[dst_v.at[pl.ds((j0 + k) * CH, CH)]],
                    sem_b, add=True
                )
                for k in range(cnt)
            ]
            ds_ = [
                pltpu.async_copy(
                    ones_v.at[pl.ds(0, CH)],
                    deg_sh.at[dst_v.at[pl.ds((j0 + k) * CH, CH)]],
                    sem_c, add=True
                )
                for k in range(cnt)
            ]
            for d in ss:
                d.wait()
            for d in ds_:
                d.wait()

        def group(g, carry):
            run_group(g * KOUT, KOUT)
            return carry

        lax.fori_loop(0, NCH // KOUT, group, 0)
        if NTAIL:
            run_group((NCH // KOUT) * KOUT, NTAIL)
        plsc.subcore_barrier()

        wb = [
            pltpu.async_copy(
                acc_sh.at[pl.ds(s * RPT, RPT)],
                out_hbm.at[c, pl.ds(s * RPT, RPT)],
                sem_a,
            ),
            pltpu.async_copy(
                deg_sh.at[pl.ds(s * RPT, RPT)],
                deg_hbm.at[c, pl.ds(s * RPT, RPT)],
                sem_b,
            ),
        ]
        for d in wb:
            d.wait()

    return run(y, edge_index)


def _head(parts, degs, b_enc, W1, b1, W2, b2, W3, b3):
    def leaky(t):
        return jnp.where(t >= 0, t, 0.01 * t)

    def body(p_ref, d_ref, be_ref, w1_ref, b1_ref, w2_ref, b2_ref, w3_ref,
             b3_ref, o_ref):
        t = p_ref[0] + p_ref[1]
        deg = jnp.maximum(d_ref[0] + d_ref[1], 1.0)
        agg = t / deg
        h = leaky(agg + be_ref[...])
        z = leaky(jnp.dot(h, w1_ref[...], preferred_element_type=jnp.float32) + b1_ref[...])
        z = leaky(jnp.dot(z, w2_ref[...], preferred_element_type=jnp.float32) + b2_ref[...])
        o_ref[...] = jnp.dot(z, w3_ref[...], preferred_element_type=jnp.float32) + b3_ref[...]

    H, C = W2.shape[0], W3.shape[1]
    return pl.pallas_call(
        body,
        grid=(N_NODES // BM,),
        in_specs=[
            pl.BlockSpec((NC, BM, H_ENC), lambda i: (0, i, 0)),
            pl.BlockSpec((NC, BM, 1), lambda i: (0, i, 0)),
            pl.BlockSpec((H_ENC,), lambda i: (0,)),
            pl.BlockSpec((H_ENC, H), lambda i: (0, 0)),
            pl.BlockSpec((H,), lambda i: (0,)),
            pl.BlockSpec((H, H), lambda i: (0, 0)),
            pl.BlockSpec((H,), lambda i: (0,)),
            pl.BlockSpec((H, C), lambda i: (0, 0)),
            pl.BlockSpec((C,), lambda i: (0,)),
        ],
        out_specs=pl.BlockSpec((BM, C), lambda i: (i, 0)),
        out_shape=jax.ShapeDtypeStruct((N_NODES, C), jnp.float32),
    )(parts, degs, b_enc, W1, b1, W2, b2, W3, b3)


def kernel(x, edge_index, W_enc, b_enc, W1, b1, W2, b2, W3, b3):
    y = _encode(x, W_enc)
    parts, degs = _sc_segment_sum(y, edge_index)
    return _head(parts, degs.reshape(NC, ACC_ROWS, 1), b_enc, W1, b1, W2, b2,
                 W3, b3)


# speedup vs baseline: 1.1536x; 1.0259x over previous
import functools

import jax
import jax.numpy as jnp
from jax import lax
from jax.experimental import pallas as pl
from jax.experimental.pallas import tpu as pltpu
from jax.experimental.pallas import tpu_sc as plsc

N_NODES = 10000
N_EDGES = 320000
F_IN = 128
H_ENC = 32

NC, NS, LANES = 2, 16, 16
NW = NC * NS
CH = 80
NCH = 125
EPW = NCH * CH
KOUT = 16
NTAIL = NCH - (NCH // KOUT) * KOUT
ACC_ROWS = 10240
RPT = ACC_ROWS // NS
ZR = 64
ONES_PAD = 112

BM = 2000


def _encode(x, W_enc):
    def body(x_ref, w_ref, o_ref):
        o_ref[...] = jnp.dot(x_ref[...], w_ref[...], preferred_element_type=jnp.float32)

    return pl.pallas_call(
        body,
        grid=(N_NODES // BM,),
        in_specs=[
            pl.BlockSpec((BM, F_IN), lambda i: (i, 0)),
            pl.BlockSpec((F_IN, H_ENC), lambda i: (0, 0)),
        ],
        out_specs=pl.BlockSpec((BM, H_ENC), lambda i: (i, 0)),
        out_shape=jax.ShapeDtypeStruct((N_NODES, H_ENC), jnp.float32),
    )(x, W_enc)


def _sc_segment_sum(y, edge_index):
    mesh = plsc.VectorSubcoreMesh(core_axis_name="c", subcore_axis_name="s")

    @functools.partial(
        pl.kernel,
        out_type=(
            jax.ShapeDtypeStruct((NC, ACC_ROWS, H_ENC), jnp.float32),
            jax.ShapeDtypeStruct((NC, ACC_ROWS), jnp.float32),
        ),
        mesh=mesh,
        scratch_types=[
            pltpu.VMEM((EPW,), jnp.int32),
            pltpu.VMEM((EPW,), jnp.int32),
            pltpu.VMEM((KOUT, CH, H_ENC), jnp.float32),
            pltpu.VMEM((ZR, H_ENC), jnp.float32),
            pltpu.VMEM((RPT,), jnp.float32),
            pltpu.VMEM((ONES_PAD,), jnp.float32),
            pltpu.VMEM_SHARED((N_NODES, H_ENC), jnp.float32),
            pltpu.VMEM_SHARED((ACC_ROWS, H_ENC), jnp.float32),
            pltpu.VMEM_SHARED((ACC_ROWS,), jnp.float32),
            pltpu.SemaphoreType.DMA,
            pltpu.SemaphoreType.DMA,
            pltpu.SemaphoreType.DMA,
        ],
        compiler_params=pltpu.CompilerParams(use_tc_tiling_on_sc=False),
    )
    def run(y_hbm, edge_hbm, out_hbm, deg_hbm, src_v, dst_v, rows_v,
            zb_v, dz_v, ones_v, y_sh, acc_sh, deg_sh, sem_a, sem_b, sem_c):
        c = lax.axis_index("c")
        s = lax.axis_index("s")
        wid = s * NC + c

        pend = [
            pltpu.async_copy(edge_hbm.at[0, pl.ds(wid * EPW, EPW)], src_v, sem_a),
            pltpu.async_copy(edge_hbm.at[1, pl.ds(wid * EPW, EPW)], dst_v, sem_a),
        ]
        ypt = N_NODES // NS
        pend.append(
            pltpu.async_copy(
                y_hbm.at[pl.ds(s * ypt, ypt)], y_sh.at[pl.ds(s * ypt, ypt)], sem_b
            )
        )
        zeros16 = jnp.zeros((LANES,), jnp.float32)
        ones16 = jnp.full((LANES,), 1.0, jnp.float32)
        for r in range(ZR):
            for g in range(H_ENC // LANES):
                zb_v[r, pl.ds(g * LANES, LANES)] = zeros16
        for r in range(RPT // LANES):
            dz_v[pl.ds(r * LANES, LANES)] = zeros16
        for r in range(ONES_PAD // LANES):
            ones_v[pl.ds(r * LANES, LANES)] = ones16
        for b in range(RPT // ZR):
            pend.append(
                pltpu.async_copy(
                    zb_v, acc_sh.at[pl.ds(s * RPT + b * ZR, ZR)], sem_c
                )
            )
        pend.append(pltpu.async_copy(dz_v, deg_sh.at[pl.ds(s * RPT, RPT)], sem_c))
        for d in pend:
            d.wait()
        plsc.subcore_barrier()

        def run_group(j0, cnt):
            gs = [
                pltpu.async_copy(
                    y_sh.at[src_v.at[pl.ds((j0 + k) * CH, CH)]], rows_v.at[k], sem_a
                )
                for k in range(cnt)
            ]
            ds_ = [
                pltpu.async_copy(
                    ones_v.at[pl.ds(0, CH)],
                    deg_sh.at[dst_v.at[pl.ds((j0 + k) * CH, CH)]],
                    sem_c, add=True
                )
                for k in range(cnt)
            ]
            for d in gs:
                d.wait()
            ss = [
                pltpu.async_copy(
                    rows_v.at[k], acc_sh.at[dst_v.at[pl.ds((j0 + k) * CH, CH)]],
                    sem_b, add=True
                )
                for k in range(cnt)
            ]
            for d in ss:
                d.wait()
            for d in ds_:
                d.wait()

        def group(g, carry):
            run_group(g * KOUT, KOUT)
            return carry

        lax.fori_loop(0, NCH // KOUT, group, 0)
        if NTAIL:
            run_group((NCH // KOUT) * KOUT, NTAIL)
        plsc.subcore_barrier()

        wb = [
            pltpu.async_copy(
                acc_sh.at[pl.ds(s * RPT, RPT)],
                out_hbm.at[c, pl.ds(s * RPT, RPT)],
                sem_a,
            ),
            pltpu.async_copy(
                deg_sh.at[pl.ds(s * RPT, RPT)],
                deg_hbm.at[c, pl.ds(s * RPT, RPT)],
                sem_b,
            ),
        ]
        for d in wb:
            d.wait()

    return run(y, edge_index)


def _head(parts, degs, b_enc, W1, b1, W2, b2, W3, b3):
    def leaky(t):
        return jnp.where(t >= 0, t, 0.01 * t)

    def body(p_ref, d_ref, be_ref, w1_ref, b1_ref, w2_ref, b2_ref, w3_ref,
             b3_ref, o_ref):
        t = p_ref[0] + p_ref[1]
        deg = jnp.maximum(d_ref[0] + d_ref[1], 1.0)
        agg = t / deg
        h = leaky(agg + be_ref[...])
        z = leaky(jnp.dot(h, w1_ref[...], preferred_element_type=jnp.float32) + b1_ref[...])
        z = leaky(jnp.dot(z, w2_ref[...], preferred_element_type=jnp.float32) + b2_ref[...])
        o_ref[...] = jnp.dot(z, w3_ref[...], preferred_element_type=jnp.float32) + b3_ref[...]

    H, C = W2.shape[0], W3.shape[1]
    return pl.pallas_call(
        body,
        grid=(N_NODES // BM,),
        in_specs=[
            pl.BlockSpec((NC, BM, H_ENC), lambda i: (0, i, 0)),
            pl.BlockSpec((NC, BM, 1), lambda i: (0, i, 0)),
            pl.BlockSpec((H_ENC,), lambda i: (0,)),
            pl.BlockSpec((H_ENC, H), lambda i: (0, 0)),
            pl.BlockSpec((H,), lambda i: (0,)),
            pl.BlockSpec((H, H), lambda i: (0, 0)),
            pl.BlockSpec((H,), lambda i: (0,)),
            pl.BlockSpec((H, C), lambda i: (0, 0)),
            pl.BlockSpec((C,), lambda i: (0,)),
        ],
        out_specs=pl.BlockSpec((BM, C), lambda i: (i, 0)),
        out_shape=jax.ShapeDtypeStruct((N_NODES, C), jnp.float32),
    )(parts, degs, b_enc, W1, b1, W2, b2, W3, b3)


def kernel(x, edge_index, W_enc, b_enc, W1, b1, W2, b2, W3, b3):
    y = _encode(x, W_enc)
    parts, degs = _sc_segment_sum(y, edge_index)
    return _head(parts, degs.reshape(NC, ACC_ROWS, 1), b_enc, W1, b1, W2, b2,
                 W3, b3)


# speedup vs baseline: 1.1570x; 1.0029x over previous
import functools

import jax
import jax.numpy as jnp
from jax import lax
from jax.experimental import pallas as pl
from jax.experimental.pallas import tpu as pltpu
from jax.experimental.pallas import tpu_sc as plsc

N_NODES = 10000
N_EDGES = 320000
F_IN = 128
H_ENC = 32

NC, NS, LANES = 2, 16, 16
NW = NC * NS
CH = 80
NCH = 125
EPW = NCH * CH
KOUT = 18
NTAIL = NCH - (NCH // KOUT) * KOUT
ACC_ROWS = 10240
RPT = ACC_ROWS // NS
ZR = 64
ONES_PAD = 112

BM = 2000


def _encode(x, W_enc):
    def body(x_ref, w_ref, o_ref):
        o_ref[...] = jnp.dot(x_ref[...], w_ref[...], preferred_element_type=jnp.float32)

    return pl.pallas_call(
        body,
        grid=(N_NODES // BM,),
        in_specs=[
            pl.BlockSpec((BM, F_IN), lambda i: (i, 0)),
            pl.BlockSpec((F_IN, H_ENC), lambda i: (0, 0)),
        ],
        out_specs=pl.BlockSpec((BM, H_ENC), lambda i: (i, 0)),
        out_shape=jax.ShapeDtypeStruct((N_NODES, H_ENC), jnp.float32),
    )(x, W_enc)


def _sc_segment_sum(y, edge_index):
    mesh = plsc.VectorSubcoreMesh(core_axis_name="c", subcore_axis_name="s")

    @functools.partial(
        pl.kernel,
        out_type=(
            jax.ShapeDtypeStruct((NC, ACC_ROWS, H_ENC), jnp.float32),
            jax.ShapeDtypeStruct((NC, ACC_ROWS), jnp.float32),
        ),
        mesh=mesh,
        scratch_types=[
            pltpu.VMEM((EPW,), jnp.int32),
            pltpu.VMEM((EPW,), jnp.int32),
            pltpu.VMEM((KOUT, CH, H_ENC), jnp.float32),
            pltpu.VMEM((ZR, H_ENC), jnp.float32),
            pltpu.VMEM((RPT,), jnp.float32),
            pltpu.VMEM((ONES_PAD,), jnp.float32),
            pltpu.VMEM_SHARED((N_NODES, H_ENC), jnp.float32),
            pltpu.VMEM_SHARED((ACC_ROWS, H_ENC), jnp.float32),
            pltpu.VMEM_SHARED((ACC_ROWS,), jnp.float32),
            pltpu.SemaphoreType.DMA,
            pltpu.SemaphoreType.DMA,
            pltpu.SemaphoreType.DMA,
        ],
        compiler_params=pltpu.CompilerParams(use_tc_tiling_on_sc=False),
    )
    def run(y_hbm, edge_hbm, out_hbm, deg_hbm, src_v, dst_v, rows_v,
            zb_v, dz_v, ones_v, y_sh, acc_sh, deg_sh, sem_a, sem_b, sem_c):
        c = lax.axis_index("c")
        s = lax.axis_index("s")
        wid = s * NC + c

        pend = [
            pltpu.async_copy(edge_hbm.at[0, pl.ds(wid * EPW, EPW)], src_v, sem_a),
            pltpu.async_copy(edge_hbm.at[1, pl.ds(wid * EPW, EPW)], dst_v, sem_a),
        ]
        ypt = N_NODES // NS
        pend.append(
            pltpu.async_copy(
                y_hbm.at[pl.ds(s * ypt, ypt)], y_sh.at[pl.ds(s * ypt, ypt)], sem_b
            )
        )
        zeros16 = jnp.zeros((LANES,), jnp.float32)
        ones16 = jnp.full((LANES,), 1.0, jnp.float32)
        for r in range(ZR):
            for g in range(H_ENC // LANES):
                zb_v[r, pl.ds(g * LANES, LANES)] = zeros16
        for r in range(RPT // LANES):
            dz_v[pl.ds(r * LANES, LANES)] = zeros16
        for r in range(ONES_PAD // LANES):
            ones_v[pl.ds(r * LANES, LANES)] = ones16
        for b in range(RPT // ZR):
            pend.append(
                pltpu.async_copy(
                    zb_v, acc_sh.at[pl.ds(s * RPT + b * ZR, ZR)], sem_c
                )
            )
        pend.append(pltpu.async_copy(dz_v, deg_sh.at[pl.ds(s * RPT, RPT)], sem_c))
        for d in pend:
            d.wait()
        plsc.subcore_barrier()

        def run_group(j0, cnt):
            gs = [
                pltpu.async_copy(
                    y_sh.at[src_v.at[pl.ds((j0 + k) * CH, CH)]], rows_v.at[k], sem_a
                )
                for k in range(cnt)
            ]
            ds_ = [
                pltpu.async_copy(
                    ones_v.at[pl.ds(0, CH)],
                    deg_sh.at[dst_v.at[pl.ds((j0 + k) * CH, CH)]],
                    sem_c, add=True
                )
                for k in range(cnt)
            ]
            for d in gs:
                d.wait()
            ss = [
                pltpu.async_copy(
                    rows_v.at[k], acc_sh.at[dst_v.at[pl.ds((j0 + k) * CH, CH)]],
                    sem_b, add=True
                )
                for k in range(cnt)
            ]
            for d in ss:
                d.wait()
            for d in ds_:
                d.wait()

        def group(g, carry):
            run_group(g * KOUT, KOUT)
            return carry

        lax.fori_loop(0, NCH // KOUT, group, 0)
        if NTAIL:
            run_group((NCH // KOUT) * KOUT, NTAIL)
        plsc.subcore_barrier()

        wb = [
            pltpu.async_copy(
                acc_sh.at[pl.ds(s * RPT, RPT)],
                out_hbm.at[c, pl.ds(s * RPT, RPT)],
                sem_a,
            ),
            pltpu.async_copy(
                deg_sh.at[pl.ds(s * RPT, RPT)],
                deg_hbm.at[c, pl.ds(s * RPT, RPT)],
                sem_b,
            ),
        ]
        for d in wb:
            d.wait()

    return run(y, edge_index)


def _head(parts, degs, b_enc, W1, b1, W2, b2, W3, b3):
    def leaky(t):
        return jnp.where(t >= 0, t, 0.01 * t)

    def body(p_ref, d_ref, be_ref, w1_ref, b1_ref, w2_ref, b2_ref, w3_ref,
             b3_ref, o_ref):
        t = p_ref[0] + p_ref[1]
        deg = jnp.maximum(d_ref[0] + d_ref[1], 1.0)
        agg = t / deg
        h = leaky(agg + be_ref[...])
        z = leaky(jnp.dot(h, w1_ref[...], preferred_element_type=jnp.float32) + b1_ref[...])
        z = leaky(jnp.dot(z, w2_ref[...], preferred_element_type=jnp.float32) + b2_ref[...])
        o_ref[...] = jnp.dot(z, w3_ref[...], preferred_element_type=jnp.float32) + b3_ref[...]

    H, C = W2.shape[0], W3.shape[1]
    return pl.pallas_call(
        body,
        grid=(N_NODES // BM,),
        in_specs=[
            pl.BlockSpec((NC, BM, H_ENC), lambda i: (0, i, 0)),
            pl.BlockSpec((NC, BM, 1), lambda i: (0, i, 0)),
            pl.BlockSpec((H_ENC,), lambda i: (0,)),
            pl.BlockSpec((H_ENC, H), lambda i: (0, 0)),
            pl.BlockSpec((H,), lambda i: (0,)),
            pl.BlockSpec((H, H), lambda i: (0, 0)),
            pl.BlockSpec((H,), lambda i: (0,)),
            pl.BlockSpec((H, C), lambda i: (0, 0)),
            pl.BlockSpec((C,), lambda i: (0,)),
        ],
        out_specs=pl.BlockSpec((BM, C), lambda i: (i, 0)),
        out_shape=jax.ShapeDtypeStruct((N_NODES, C), jnp.float32),
    )(parts, degs, b_enc, W1, b1, W2, b2, W3, b3)


def kernel(x, edge_index, W_enc, b_enc, W1, b1, W2, b2, W3, b3):
    y = _encode(x, W_enc)
    parts, degs = _sc_segment_sum(y, edge_index)
    return _head(parts, degs.reshape(NC, ACC_ROWS, 1), b_enc, W1, b1, W2, b2,
                 W3, b3)
